# baseline (speedup 1.0000x reference)
import functools

import jax
import jax.numpy as jnp
from jax import lax
from jax.experimental import pallas as pl
from jax.experimental.pallas import tpu as pltpu, tpu_sc as plsc

NUM_CORES = 2
NUM_SUBCORES = 16
NUM_WORKERS = NUM_CORES * NUM_SUBCORES
L = 16

B = 16384
D = 64
V = 1000000
R = 1000
NCOLS = 7813
CPT = 256
SUP = 2
NSUP = CPT // SUP
HSZ = 144
REQ_CAP = 4096
STAGE = 2 * B + L
BPW = B // NUM_WORKERS


def _extract_body(si_hbm, oi_hbm, nt_hbm, stage_hbm,
                  idx_v, req_i, req_t, req2_i, req2_t, s_tmp,
                  hist, starts, cursors,
                  slab, outbuf, tagbuf, tagidx,
                  sem0, sem1, sem2, sem3, scsem):
    wid = lax.axis_index("s") * NUM_CORES + lax.axis_index("c")
    lo_col = wid * CPT
    lane = lax.iota(jnp.int32, L)
    sink = jnp.full((L,), 2 * B, jnp.int32) + lane
    sems = [sem0, sem1, sem2, sem3]

    def scan_src(src_hbm, tag_base, off0):
        pltpu.sync_copy(src_hbm, idx_v)

        def chunk(k, off):
            v = idx_v[pl.ds(k * L, L)]
            col = lax.shift_right_logical(v, 7)
            m = (col >= lo_col) & (col < lo_col + CPT)
            cnt = plsc.all_reduce_population_count(m)[0]
            offc = jnp.minimum(off, REQ_CAP - L)
            plsc.store_compressed(req_i.at[pl.ds(offc, L)], v, mask=m)
            plsc.store_compressed(req_t.at[pl.ds(offc, L)],
                                  tag_base + k * L + lane, mask=m)
            return jnp.minimum(off + cnt, REQ_CAP - L)
        return lax.fori_loop(0, B // L, chunk, off0)

    n_req = scan_src(si_hbm, 0, 0)
    n_req = scan_src(oi_hbm, B, n_req)
    n_chunks = (n_req + L - 1) // L

    for q in range(12):
        tagbuf[pl.ds(q * L, L)] = sink

    def fetch_super(s_idx, slot):
        scol = lo_col + s_idx * SUP

        @pl.when(scol < NCOLS)
        def _():
            sbase = jnp.minimum(scol, NCOLS - SUP)
            off = pl.multiple_of(sbase * 128, 128)
            pltpu.async_copy(nt_hbm.at[:, pl.ds(off, SUP * 128)],
                             slab.at[slot], sems[slot])

    def drain_super(s_idx, slot):
        scol = lo_col + s_idx * SUP

        @pl.when(scol < NCOLS)
        def _():
            pltpu.make_async_copy(nt_hbm.at[:, pl.ds(0, SUP * 128)],
                                  slab.at[slot], sems[slot]).wait()

    def flush128():
        for q in range(8):
            tagidx[q, pl.ds(0, L)] = tagbuf[pl.ds(q * L, L)]
        cps = []
        for q in range(8):
            cps.append(pltpu.make_async_copy(
                outbuf.at[pl.ds(q * L, L), :],
                stage_hbm.at[tagidx.at[q]], scsem))
        for cp in cps:
            cp.start()
        for cp in cps:
            cp.wait()
        for rr in range(L):
            for j in range(4):
                outbuf[rr, pl.ds(j * L, L)] = outbuf[128 + rr,
                                                     pl.ds(j * L, L)]
        tagbuf[pl.ds(0, L)] = tagbuf[pl.ds(128, L)]
        for q in range(1, 12):
            tagbuf[pl.ds(q * L, L)] = sink

    zeros16 = jnp.zeros((L,), jnp.int32)

    def process_super(s_idx, slot, ob):
        scol = lo_col + s_idx * SUP
        sbase = jnp.minimum(scol, NCOLS - SUP)

        def req_chunk(k, ob):
            v = req_i[pl.ds(k * L, L)]
            t = req_t[pl.ds(k * L, L)]
            col = lax.shift_right_logical(v, 7)
            valid = (k * L + lane) < n_req
            m = (col >= scol) & (col < scol + SUP) & valid
            cnt = plsc.all_reduce_population_count(m)[0]

            def do_extract(ob):
                hist[pl.ds(0, L)] = zeros16
                starts[pl.ds(0, L)] = sink
                plsc.store_compressed(hist.at[pl.ds(0, L)], v, mask=m)
                plsc.store_compressed(starts.at[pl.ds(0, L)], t, mask=m)
                ti = hist[pl.ds(0, L)]
                tt = starts[pl.ds(0, L)]
                tagbuf[pl.ds(ob, L)] = tt
                for r in range(L):
                    i_r = ti[r]
                    slot_r = jnp.clip(
                        lax.shift_right_logical(i_r, 7) - sbase,
                        0, SUP - 1)
                    il = i_r & 127
                    il_s = jnp.full((L,), slot_r * 128 + il, jnp.int32)
                    for j in range(4):
                        rows = j * L + lane
                        q16 = plsc.load_gather(slab.at[slot],
                                               [rows, il_s])
                        outbuf[ob + r, pl.ds(j * L, L)] = q16
                return ob + cnt

            ob = lax.cond(cnt > 0, do_extract, lambda ob: ob, ob)

            def do_flush(ob):
                flush128()
                return ob - 128
            return lax.cond(ob >= 128, do_flush, lambda ob: ob, ob)

        return lax.fori_loop(0, n_chunks, req_chunk, ob)

    for s in range(3):
        fetch_super(s, s)

    def quad(u, ob):
        for k in range(4):
            s = 4 * u + k

            @pl.when(s + 3 < NSUP)
            def _():
                fetch_super(s + 3, (k + 3) % 4)
            drain_super(s, k)
            ob = process_super(s, k, ob)
        return ob

    out_base = lax.fori_loop(0, NSUP // 4, quad, 0)

    for q in range(8):
        tagidx[q, pl.ds(0, L)] = tagbuf[pl.ds(q * L, L)]
    cps = []
    for q in range(8):
        cps.append(pltpu.make_async_copy(
            outbuf.at[pl.ds(q * L, L), :],
            stage_hbm.at[tagidx.at[q]], scsem))
    for q in range(8):
        @pl.when(q * L < out_base)
        def _():
            cps[q].start()
    for q in range(8):
        @pl.when(q * L < out_base)
        def _():
            cps[q].wait()


def _score_body(stage_hbm, pi_hbm, rt_hbm, out_hbm,
                pi_v, rt_v, s_v, o_v, out_v, shared):
    cid = lax.axis_index("c")
    sid = lax.axis_index("s")
    wid = cid * NUM_SUBCORES + sid
    base = wid * BPW
    lane = lax.iota(jnp.int32, L)
    pltpu.sync_copy(pi_hbm, pi_v)
    pltpu.sync_copy(rt_hbm, rt_v)

    def one_pass(p, _):
        pltpu.sync_copy(stage_hbm.at[pl.ds(base + p * 128, 128), :], s_v)
        pltpu.sync_copy(stage_hbm.at[pl.ds(B + base + p * 128, 128), :],
                        o_v)

        def group(g, _):
            pv = pi_v[pl.ds(base + p * 128 + g * L, L)]
            out16 = jnp.zeros((L,), jnp.float32)
            for r in range(L):
                b = g * L + r
                pv_s = jnp.full((L,), pv[r], jnp.int32)
                acc = jnp.zeros((L,), jnp.float32)
                for j in range(4):
                    dj = j * L + lane
                    pq = plsc.load_gather(rt_v, [dj, pv_s])
                    acc = acc + (s_v[b, pl.ds(j * L, L)]
                                 * o_v[b, pl.ds(j * L, L)] * pq)
                out16 = jnp.where(lane == r, jnp.sum(acc), out16)
            out_v[pl.ds(p * 128 + g * L, L)] = out16
            return 0
        lax.fori_loop(0, 8, group, 0)
        return 0

    lax.fori_loop(0, BPW // 128, one_pass, 0)

    pltpu.sync_copy(out_v, shared.at[pl.ds(sid * BPW, BPW)])
    plsc.subcore_barrier()

    @pl.when(sid == 0)
    def _():
        pltpu.sync_copy(shared,
                        out_hbm.at[pl.ds(cid * NUM_SUBCORES * BPW,
                                         NUM_SUBCORES * BPW)])


@jax.jit
def _distmult(si, pi, oi, nodes, relations):
    mesh = plsc.VectorSubcoreMesh(core_axis_name="c", subcore_axis_name="s")
    nt = jnp.swapaxes(nodes, 0, 1)
    rt = jnp.swapaxes(jnp.pad(relations, ((0, 1024 - R), (0, 0))), 0, 1)
    cp = pltpu.CompilerParams(needs_layout_passes=False)

    stage = pl.kernel(
        _extract_body,
        out_type=jax.ShapeDtypeStruct((STAGE, 128), jnp.float32),
        mesh=mesh,
        scratch_types=[
            pltpu.VMEM((B,), jnp.int32),
            pltpu.VMEM((REQ_CAP,), jnp.int32),
            pltpu.VMEM((REQ_CAP,), jnp.int32),
            pltpu.VMEM((REQ_CAP,), jnp.int32),
            pltpu.VMEM((REQ_CAP,), jnp.int32),
            pltpu.VMEM((L,), jnp.int32),
            pltpu.VMEM((HSZ,), jnp.int32),
            pltpu.VMEM((HSZ,), jnp.int32),
            pltpu.VMEM((HSZ,), jnp.int32),
            pltpu.VMEM((4, D, SUP * 128), jnp.float32),
            pltpu.VMEM((192, 128), jnp.float32),
            pltpu.VMEM((192,), jnp.int32),
            pltpu.VMEM((8, L), jnp.int32),
            pltpu.SemaphoreType.DMA,
            pltpu.SemaphoreType.DMA,
            pltpu.SemaphoreType.DMA,
            pltpu.SemaphoreType.DMA,
            pltpu.SemaphoreType.DMA,
        ],
        compiler_params=cp,
    )(si, oi, nt)

    return pl.kernel(
        _score_body,
        out_type=jax.ShapeDtypeStruct((B,), jnp.float32),
        mesh=mesh,
        scratch_types=[
            pltpu.VMEM((B,), jnp.int32),
            pltpu.VMEM((D, 1024), jnp.float32),
            pltpu.VMEM((128, 128), jnp.float32),
            pltpu.VMEM((128, 128), jnp.float32),
            pltpu.VMEM((BPW,), jnp.float32),
            pltpu.VMEM_SHARED((NUM_SUBCORES * BPW,), jnp.float32),
        ],
        compiler_params=cp,
    )(stage, pi, rt)


def kernel(si, pi, oi, nodes, relations):
    return _distmult(si.astype(jnp.int32), pi.astype(jnp.int32),
                     oi.astype(jnp.int32), nodes, relations)


# speedup vs baseline: 2.0216x; 2.0216x over previous
import functools

import jax
import jax.numpy as jnp
from jax import lax
from jax.experimental import pallas as pl
from jax.experimental.pallas import tpu as pltpu, tpu_sc as plsc

NUM_CORES = 2
NUM_SUBCORES = 16
NUM_WORKERS = NUM_CORES * NUM_SUBCORES
L = 16

B = 16384
D = 64
V = 1000000
R = 1000
NCOLS = 7813
CPT = 256
SUP = 2
NSUP = CPT // SUP
GCAP = 512
REQ_CAP = 4096
STAGE = 2 * B + L
BPW = B // NUM_WORKERS


def _extract_body(si_hbm, oi_hbm, nt_hbm, stage_hbm,
                  idx_v, req_i, req_t, req2_i, req2_t,
                  gcnt_v, tmp_i, tmp_t,
                  slab, outbuf, tagbuf, tagidx,
                  sem0, sem1, sem2, sem3, scsem):
    wid = lax.axis_index("s") * NUM_CORES + lax.axis_index("c")
    lo_col = wid * CPT
    lane = lax.iota(jnp.int32, L)
    sink = jnp.full((L,), 2 * B, jnp.int32) + lane
    sems = [sem0, sem1, sem2, sem3]

    def scan_src(src_hbm, tag_base, off0):
        pltpu.sync_copy(src_hbm, idx_v)

        def chunk8(k8, off):
            for kk in range(8):
                k = k8 * 8 + kk
                v = idx_v[pl.ds(k * L, L)]
                col = lax.shift_right_logical(v, 7)
                m = (col >= lo_col) & (col < lo_col + CPT)
                cnt = plsc.all_reduce_population_count(m)[0]
                offc = jnp.minimum(off, REQ_CAP - L)
                plsc.store_compressed(req_i.at[pl.ds(offc, L)], v, mask=m)
                plsc.store_compressed(req_t.at[pl.ds(offc, L)],
                                      tag_base + k * L + lane, mask=m)
                off = jnp.minimum(off + cnt, REQ_CAP - L)
            return off
        return lax.fori_loop(0, B // L // 8, chunk8, off0)

    n_req = scan_src(si_hbm, 0, 0)
    n_req = scan_src(oi_hbm, B, n_req)
    n_chunks = (n_req + L - 1) // L

    zeros16 = jnp.zeros((L,), jnp.int32)
    gcounts = zeros16
    for g in range(16):
        def gscan(k4, cur, g=g):
            for kk in range(4):
                k = k4 * 4 + kk
                v = req_i[pl.ds(k * L, L)]
                t = req_t[pl.ds(k * L, L)]
                col = lax.shift_right_logical(v, 7)
                m = (lax.shift_right_logical(col - lo_col, 4) == g)
                m = m & ((k * L + lane) < n_req)
                cnt = plsc.all_reduce_population_count(m)[0]
                curc = jnp.minimum(cur, GCAP - L)
                plsc.store_compressed(req2_i.at[g, pl.ds(curc, L)],
                                      v, mask=m)
                plsc.store_compressed(req2_t.at[g, pl.ds(curc, L)],
                                      t, mask=m)
                cur = jnp.minimum(cur + cnt, GCAP - L)
            return cur
        cur_g = lax.fori_loop(0, (REQ_CAP // L) // 4, gscan, 0)
        gcounts = jnp.where(lane == g, cur_g, gcounts)
    gcnt_v[pl.ds(0, L)] = gcounts

    for q in range(12):
        tagbuf[pl.ds(q * L, L)] = sink

    def fetch_super(s_idx, slot):
        scol = lo_col + s_idx * SUP

        @pl.when(scol < NCOLS)
        def _():
            sbase = jnp.minimum(scol, NCOLS - SUP)
            off = pl.multiple_of(sbase * 128, 128)
            pltpu.async_copy(nt_hbm.at[:, pl.ds(off, SUP * 128)],
                             slab.at[slot], sems[slot])

    def drain_super(s_idx, slot):
        scol = lo_col + s_idx * SUP

        @pl.when(scol < NCOLS)
        def _():
            pltpu.make_async_copy(nt_hbm.at[:, pl.ds(0, SUP * 128)],
                                  slab.at[slot], sems[slot]).wait()

    def flush128():
        for q in range(8):
            tagidx[q, pl.ds(0, L)] = tagbuf[pl.ds(q * L, L)]
        cps = []
        for q in range(8):
            cps.append(pltpu.make_async_copy(
                outbuf.at[pl.ds(q * L, L), :],
                stage_hbm.at[tagidx.at[q]], scsem))
        for cp in cps:
            cp.start()
        for cp in cps:
            cp.wait()
        for rr in range(L):
            for j in range(4):
                outbuf[rr, pl.ds(j * L, L)] = outbuf[128 + rr,
                                                     pl.ds(j * L, L)]
        tagbuf[pl.ds(0, L)] = tagbuf[pl.ds(128, L)]
        for q in range(1, 12):
            tagbuf[pl.ds(q * L, L)] = sink

    def process_super(s_idx, slot, ob):
        scol = lo_col + s_idx * SUP
        sbase = jnp.minimum(scol, NCOLS - SUP)
        g = lax.shift_right_logical(s_idx, 3)
        cnt_g = plsc.load_gather(gcnt_v,
                                 [jnp.full((L,), g, jnp.int32)])[0]

        def req_chunk(k, ob):
            v = req2_i[g, pl.ds(k * L, L)]
            t = req2_t[g, pl.ds(k * L, L)]
            col = lax.shift_right_logical(v, 7)
            m = (col >= scol) & (col < scol + SUP)
            m = m & ((k * L + lane) < cnt_g)
            cnt = plsc.all_reduce_population_count(m)[0]

            def do_extract(ob):
                tmp_i[pl.ds(0, L)] = jnp.zeros((L,), jnp.int32)
                tmp_t[pl.ds(0, L)] = sink
                plsc.store_compressed(tmp_i.at[pl.ds(0, L)], v, mask=m)
                plsc.store_compressed(tmp_t.at[pl.ds(0, L)], t, mask=m)
                ti = tmp_i[pl.ds(0, L)]
                tt = tmp_t[pl.ds(0, L)]
                tagbuf[pl.ds(ob, L)] = tt
                for r in range(L):
                    i_r = ti[r]
                    slot_r = jnp.clip(
                        lax.shift_right_logical(i_r, 7) - sbase,
                        0, SUP - 1)
                    il = i_r & 127
                    il_s = jnp.full((L,), slot_r * 128 + il, jnp.int32)
                    for j in range(4):
                        rows = j * L + lane
                        q16 = plsc.load_gather(slab.at[slot],
                                               [rows, il_s])
                        outbuf[ob + r, pl.ds(j * L, L)] = q16
                return ob + cnt

            ob = lax.cond(cnt > 0, do_extract, lambda ob: ob, ob)

            def do_flush(ob):
                flush128()
                return ob - 128
            return lax.cond(ob >= 128, do_flush, lambda ob: ob, ob)

        return lax.fori_loop(0, (cnt_g + L - 1) // L, req_chunk, ob)

    for s in range(3):
        fetch_super(s, s)

    def quad(u, ob):
        for k in range(4):
            s = 4 * u + k

            @pl.when(s + 3 < NSUP)
            def _():
                fetch_super(s + 3, (k + 3) % 4)
            drain_super(s, k)
            ob = process_super(s, k, ob)
        return ob

    out_base = lax.fori_loop(0, NSUP // 4, quad, 0)

    for q in range(8):
        tagidx[q, pl.ds(0, L)] = tagbuf[pl.ds(q * L, L)]
    cps = []
    for q in range(8):
        cps.append(pltpu.make_async_copy(
            outbuf.at[pl.ds(q * L, L), :],
            stage_hbm.at[tagidx.at[q]], scsem))
    for q in range(8):
        @pl.when(q * L < out_base)
        def _():
            cps[q].start()
    for q in range(8):
        @pl.when(q * L < out_base)
        def _():
            cps[q].wait()


def _score_body(stage_hbm, pi_hbm, rt_hbm, out_hbm,
                pi_v, rt_v, s_v, o_v, out_v, shared):
    cid = lax.axis_index("c")
    sid = lax.axis_index("s")
    wid = cid * NUM_SUBCORES + sid
    base = wid * BPW
    lane = lax.iota(jnp.int32, L)
    pltpu.sync_copy(pi_hbm, pi_v)
    pltpu.sync_copy(rt_hbm, rt_v)

    def one_pass(p, _):
        pltpu.sync_copy(stage_hbm.at[pl.ds(base + p * 128, 128), :], s_v)
        pltpu.sync_copy(stage_hbm.at[pl.ds(B + base + p * 128, 128), :],
                        o_v)

        def group(g, _):
            pv = pi_v[pl.ds(base + p * 128 + g * L, L)]
            out16 = jnp.zeros((L,), jnp.float32)
            for r in range(L):
                b = g * L + r
                pv_s = jnp.full((L,), pv[r], jnp.int32)
                acc = jnp.zeros((L,), jnp.float32)
                for j in range(4):
                    dj = j * L + lane
                    pq = plsc.load_gather(rt_v, [dj, pv_s])
                    acc = acc + (s_v[b, pl.ds(j * L, L)]
                                 * o_v[b, pl.ds(j * L, L)] * pq)
                out16 = jnp.where(lane == r, jnp.sum(acc), out16)
            out_v[pl.ds(p * 128 + g * L, L)] = out16
            return 0
        lax.fori_loop(0, 8, group, 0)
        return 0

    lax.fori_loop(0, BPW // 128, one_pass, 0)

    pltpu.sync_copy(out_v, shared.at[pl.ds(sid * BPW, BPW)])
    plsc.subcore_barrier()

    @pl.when(sid == 0)
    def _():
        pltpu.sync_copy(shared,
                        out_hbm.at[pl.ds(cid * NUM_SUBCORES * BPW,
                                         NUM_SUBCORES * BPW)])


@jax.jit
def _distmult(si, pi, oi, nodes, relations):
    mesh = plsc.VectorSubcoreMesh(core_axis_name="c", subcore_axis_name="s")
    nt = jnp.swapaxes(nodes, 0, 1)
    rt = jnp.swapaxes(jnp.pad(relations, ((0, 1024 - R), (0, 0))), 0, 1)
    cp = pltpu.CompilerParams(needs_layout_passes=False)

    stage = pl.kernel(
        _extract_body,
        out_type=jax.ShapeDtypeStruct((STAGE, 128), jnp.float32),
        mesh=mesh,
        scratch_types=[
            pltpu.VMEM((B,), jnp.int32),
            pltpu.VMEM((REQ_CAP,), jnp.int32),
            pltpu.VMEM((REQ_CAP,), jnp.int32),
            pltpu.VMEM((16, GCAP), jnp.int32),
            pltpu.VMEM((16, GCAP), jnp.int32),
            pltpu.VMEM((L,), jnp.int32),
            pltpu.VMEM((L,), jnp.int32),
            pltpu.VMEM((L,), jnp.int32),
            pltpu.VMEM((4, D, SUP * 128), jnp.float32),
            pltpu.VMEM((176, 128), jnp.float32),
            pltpu.VMEM((192,), jnp.int32),
            pltpu.VMEM((8, L), jnp.int32),
            pltpu.SemaphoreType.DMA,
            pltpu.SemaphoreType.DMA,
            pltpu.SemaphoreType.DMA,
            pltpu.SemaphoreType.DMA,
            pltpu.SemaphoreType.DMA,
        ],
        compiler_params=cp,
    )(si, oi, nt)

    return pl.kernel(
        _score_body,
        out_type=jax.ShapeDtypeStruct((B,), jnp.float32),
        mesh=mesh,
        scratch_types=[
            pltpu.VMEM((B,), jnp.int32),
            pltpu.VMEM((D, 1024), jnp.float32),
            pltpu.VMEM((128, 128), jnp.float32),
            pltpu.VMEM((128, 128), jnp.float32),
            pltpu.VMEM((BPW,), jnp.float32),
            pltpu.VMEM_SHARED((NUM_SUBCORES * BPW,), jnp.float32),
        ],
        compiler_params=cp,
    )(stage, pi, rt)


def kernel(si, pi, oi, nodes, relations):
    return _distmult(si.astype(jnp.int32), pi.astype(jnp.int32),
                     oi.astype(jnp.int32), nodes, relations)


# speedup vs baseline: 2.9507x; 1.4596x over previous
import functools

import jax
import jax.numpy as jnp
from jax import lax
from jax.experimental import pallas as pl
from jax.experimental.pallas import tpu as pltpu, tpu_sc as plsc

NUM_CORES = 2
NUM_SUBCORES = 16
NUM_WORKERS = NUM_CORES * NUM_SUBCORES
L = 16

B = 16384
D = 64
V = 1000000
R = 1000
NCOLS = 7813
CPT = 256
SUP = 4
NSUP = CPT // SUP
GCAP = 512
REQ_CAP = 4096
STAGE = 2 * B + L
BPW = B // NUM_WORKERS


def _extract_body(si_hbm, oi_hbm, nt_hbm, stage_hbm,
                  idx_v, req_i, req_t, req2_i, req2_t,
                  gcnt_v, tmp_i, tmp_t,
                  slab, outbuf, tagbuf, tagidx,
                  sem0, sem1, sem2, sem3, scsem):
    wid = lax.axis_index("s") * NUM_CORES + lax.axis_index("c")
    lo_col = wid * CPT
    lane = lax.iota(jnp.int32, L)
    sink = jnp.full((L,), 2 * B, jnp.int32) + lane
    sems = [sem0, sem1, sem2, sem3]

    def scan_src(src_hbm, tag_base, off0):
        pltpu.sync_copy(src_hbm, idx_v)

        def chunk8(k8, off):
            for kk in range(8):
                k = k8 * 8 + kk
                v = idx_v[pl.ds(k * L, L)]
                col = lax.shift_right_logical(v, 7)
                m = (col >= lo_col) & (col < lo_col + CPT)
                cnt = plsc.all_reduce_population_count(m)[0]
                offc = jnp.minimum(off, REQ_CAP - L)
                plsc.store_compressed(req_i.at[pl.ds(offc, L)], v, mask=m)
                plsc.store_compressed(req_t.at[pl.ds(offc, L)],
                                      tag_base + k * L + lane, mask=m)
                off = jnp.minimum(off + cnt, REQ_CAP - L)
            return off
        return lax.fori_loop(0, B // L // 8, chunk8, off0)

    n_req = scan_src(si_hbm, 0, 0)
    n_req = scan_src(oi_hbm, B, n_req)
    n_chunks = (n_req + L - 1) // L

    zeros16 = jnp.zeros((L,), jnp.int32)
    gcounts = zeros16
    for g in range(16):
        def gscan(k4, cur, g=g):
            for kk in range(4):
                k = k4 * 4 + kk
                v = req_i[pl.ds(k * L, L)]
                t = req_t[pl.ds(k * L, L)]
                col = lax.shift_right_logical(v, 7)
                m = (lax.shift_right_logical(col - lo_col, 4) == g)
                m = m & ((k * L + lane) < n_req)
                cnt = plsc.all_reduce_population_count(m)[0]
                curc = jnp.minimum(cur, GCAP - L)
                plsc.store_compressed(req2_i.at[g, pl.ds(curc, L)],
                                      v, mask=m)
                plsc.store_compressed(req2_t.at[g, pl.ds(curc, L)],
                                      t, mask=m)
                cur = jnp.minimum(cur + cnt, GCAP - L)
            return cur
        cur_g = lax.fori_loop(0, (REQ_CAP // L) // 4, gscan, 0)
        gcounts = jnp.where(lane == g, cur_g, gcounts)
    gcnt_v[pl.ds(0, L)] = gcounts

    for q in range(12):
        tagbuf[pl.ds(q * L, L)] = sink

    def fetch_super(s_idx, slot):
        scol = lo_col + s_idx * SUP

        @pl.when(scol < NCOLS)
        def _():
            sbase = jnp.minimum(scol, NCOLS - SUP)
            off = pl.multiple_of(sbase * 128, 128)
            pltpu.async_copy(nt_hbm.at[:, pl.ds(off, SUP * 128)],
                             slab.at[slot], sems[slot])

    def drain_super(s_idx, slot):
        scol = lo_col + s_idx * SUP

        @pl.when(scol < NCOLS)
        def _():
            pltpu.make_async_copy(nt_hbm.at[:, pl.ds(0, SUP * 128)],
                                  slab.at[slot], sems[slot]).wait()

    def flush128():
        for q in range(8):
            tagidx[q, pl.ds(0, L)] = tagbuf[pl.ds(q * L, L)]
        cps = []
        for q in range(8):
            cps.append(pltpu.make_async_copy(
                outbuf.at[pl.ds(q * L, L), :],
                stage_hbm.at[tagidx.at[q]], scsem))
        for cp in cps:
            cp.start()
        for cp in cps:
            cp.wait()
        for rr in range(L):
            for j in range(4):
                outbuf[rr, pl.ds(j * L, L)] = outbuf[128 + rr,
                                                     pl.ds(j * L, L)]
        tagbuf[pl.ds(0, L)] = tagbuf[pl.ds(128, L)]
        for q in range(1, 12):
            tagbuf[pl.ds(q * L, L)] = sink

    def process_super(s_idx, slot, ob):
        scol = lo_col + s_idx * SUP
        sbase = jnp.minimum(scol, NCOLS - SUP)
        g = lax.shift_right_logical(s_idx, 2)
        cnt_g = plsc.load_gather(gcnt_v,
                                 [jnp.full((L,), g, jnp.int32)])[0]

        def req_chunk(k, ob):
            v = req2_i[g, pl.ds(k * L, L)]
            t = req2_t[g, pl.ds(k * L, L)]
            col = lax.shift_right_logical(v, 7)
            m = (col >= scol) & (col < scol + SUP)
            m = m & ((k * L + lane) < cnt_g)
            cnt = plsc.all_reduce_population_count(m)[0]

            def do_extract(ob):
                tmp_i[pl.ds(0, L)] = jnp.zeros((L,), jnp.int32)
                tmp_t[pl.ds(0, L)] = sink
                plsc.store_compressed(tmp_i.at[pl.ds(0, L)], v, mask=m)
                plsc.store_compressed(tmp_t.at[pl.ds(0, L)], t, mask=m)
                ti = tmp_i[pl.ds(0, L)]
                tt = tmp_t[pl.ds(0, L)]
                tagbuf[pl.ds(ob, L)] = tt
                for r in range(L):
                    i_r = ti[r]
                    slot_r = jnp.clip(
                        lax.shift_right_logical(i_r, 7) - sbase,
                        0, SUP - 1)
                    il = i_r & 127
                    il_s = jnp.full((L,), slot_r * 128 + il, jnp.int32)
                    for j in range(4):
                        rows = j * L + lane
                        q16 = plsc.load_gather(slab.at[slot],
                                               [rows, il_s])
                        outbuf[ob + r, pl.ds(j * L, L)] = q16
                return ob + cnt

            ob = lax.cond(cnt > 0, do_extract, lambda ob: ob, ob)

            def do_flush(ob):
                flush128()
                return ob - 128
            return lax.cond(ob >= 128, do_flush, lambda ob: ob, ob)

        return lax.fori_loop(0, (cnt_g + L - 1) // L, req_chunk, ob)

    fetch_super(0, 0)

    def pair(u, ob):
        s = 2 * u

        @pl.when(s + 1 < NSUP)
        def _():
            fetch_super(s + 1, 1)
        drain_super(s, 0)
        ob = process_super(s, 0, ob)

        @pl.when(s + 2 < NSUP)
        def _():
            fetch_super(s + 2, 0)
        drain_super(s + 1, 1)
        ob = process_super(s + 1, 1, ob)
        return ob

    out_base = lax.fori_loop(0, NSUP // 2, pair, 0)

    for q in range(8):
        tagidx[q, pl.ds(0, L)] = tagbuf[pl.ds(q * L, L)]
    cps = []
    for q in range(8):
        cps.append(pltpu.make_async_copy(
            outbuf.at[pl.ds(q * L, L), :],
            stage_hbm.at[tagidx.at[q]], scsem))
    for q in range(8):
        @pl.when(q * L < out_base)
        def _():
            cps[q].start()
    for q in range(8):
        @pl.when(q * L < out_base)
        def _():
            cps[q].wait()


def _score_body(stage_hbm, pi_hbm, rt_hbm, out_hbm,
                pi_v, rt_v, s_v, o_v, out_v, shared):
    cid = lax.axis_index("c")
    sid = lax.axis_index("s")
    wid = cid * NUM_SUBCORES + sid
    base = wid * BPW
    lane = lax.iota(jnp.int32, L)
    pltpu.sync_copy(pi_hbm, pi_v)
    pltpu.sync_copy(rt_hbm, rt_v)

    def one_pass(p, _):
        pltpu.sync_copy(stage_hbm.at[pl.ds(base + p * 128, 128), :], s_v)
        pltpu.sync_copy(stage_hbm.at[pl.ds(B + base + p * 128, 128), :],
                        o_v)

        def group(g, _):
            pv = pi_v[pl.ds(base + p * 128 + g * L, L)]
            out16 = jnp.zeros((L,), jnp.float32)
            for r in range(L):
                b = g * L + r
                pv_s = jnp.full((L,), pv[r], jnp.int32)
                acc = jnp.zeros((L,), jnp.float32)
                for j in range(4):
                    dj = j * L + lane
                    pq = plsc.load_gather(rt_v, [dj, pv_s])
                    acc = acc + (s_v[b, pl.ds(j * L, L)]
                                 * o_v[b, pl.ds(j * L, L)] * pq)
                out16 = jnp.where(lane == r, jnp.sum(acc), out16)
            out_v[pl.ds(p * 128 + g * L, L)] = out16
            return 0
        lax.fori_loop(0, 8, group, 0)
        return 0

    lax.fori_loop(0, BPW // 128, one_pass, 0)

    pltpu.sync_copy(out_v, shared.at[pl.ds(sid * BPW, BPW)])
    plsc.subcore_barrier()

    @pl.when(sid == 0)
    def _():
        pltpu.sync_copy(shared,
                        out_hbm.at[pl.ds(cid * NUM_SUBCORES * BPW,
                                         NUM_SUBCORES * BPW)])


@jax.jit
def _distmult(si, pi, oi, nodes, relations):
    mesh = plsc.VectorSubcoreMesh(core_axis_name="c", subcore_axis_name="s")
    nt = jnp.swapaxes(nodes, 0, 1)
    rt = jnp.swapaxes(jnp.pad(relations, ((0, 1024 - R), (0, 0))), 0, 1)
    cp = pltpu.CompilerParams(needs_layout_passes=False)

    stage = pl.kernel(
        _extract_body,
        out_type=jax.ShapeDtypeStruct((STAGE, 128), jnp.float32),
        mesh=mesh,
        scratch_types=[
            pltpu.VMEM((B,), jnp.int32),
            pltpu.VMEM((REQ_CAP,), jnp.int32),
            pltpu.VMEM((REQ_CAP,), jnp.int32),
            pltpu.VMEM((16, GCAP), jnp.int32),
            pltpu.VMEM((16, GCAP), jnp.int32),
            pltpu.VMEM((L,), jnp.int32),
            pltpu.VMEM((L,), jnp.int32),
            pltpu.VMEM((L,), jnp.int32),
            pltpu.VMEM((2, D, SUP * 128), jnp.float32),
            pltpu.VMEM((176, 128), jnp.float32),
            pltpu.VMEM((192,), jnp.int32),
            pltpu.VMEM((8, L), jnp.int32),
            pltpu.SemaphoreType.DMA,
            pltpu.SemaphoreType.DMA,
            pltpu.SemaphoreType.DMA,
            pltpu.SemaphoreType.DMA,
            pltpu.SemaphoreType.DMA,
        ],
        compiler_params=cp,
    )(si, oi, nt)

    return pl.kernel(
        _score_body,
        out_type=jax.ShapeDtypeStruct((B,), jnp.float32),
        mesh=mesh,
        scratch_types=[
            pltpu.VMEM((B,), jnp.int32),
            pltpu.VMEM((D, 1024), jnp.float32),
            pltpu.VMEM((128, 128), jnp.float32),
            pltpu.VMEM((128, 128), jnp.float32),
            pltpu.VMEM((BPW,), jnp.float32),
            pltpu.VMEM_SHARED((NUM_SUBCORES * BPW,), jnp.float32),
        ],
        compiler_params=cp,
    )(stage, pi, rt)


def kernel(si, pi, oi, nodes, relations):
    return _distmult(si.astype(jnp.int32), pi.astype(jnp.int32),
                     oi.astype(jnp.int32), nodes, relations)


# speedup vs baseline: 3.2111x; 1.0883x over previous
import functools

import jax
import jax.numpy as jnp
from jax import lax
from jax.experimental import pallas as pl
from jax.experimental.pallas import tpu as pltpu, tpu_sc as plsc

NUM_CORES = 2
NUM_SUBCORES = 16
NUM_WORKERS = NUM_CORES * NUM_SUBCORES
L = 16

B = 16384
D = 64
V = 1000000
R = 1000
NCOLS = 7813
CPT = 256
SUP = 4
NSUP = CPT // SUP
GCAP = 512
REQ_CAP = 4096
STAGE = 2 * B + L
BPW = B // NUM_WORKERS


def _extract_body(si_hbm, oi_hbm, nt_hbm, stage_hbm,
                  idx_v, req_i, req_t, req2_i, req2_t,
                  gcnt_v, tmp_i, tmp_t,
                  slab, outbuf, tagbuf, tagidx,
                  sem0, sem1, sem2, sem3, scsem):
    wid = lax.axis_index("s") * NUM_CORES + lax.axis_index("c")
    lo_col = wid * CPT
    lane = lax.iota(jnp.int32, L)
    sink = jnp.full((L,), 2 * B, jnp.int32) + lane
    sems = [sem0, sem1, sem2, sem3]

    def scan_src(src_hbm, tag_base, off0):
        pltpu.sync_copy(src_hbm, idx_v)

        def chunk8(k8, off):
            for kk in range(8):
                k = k8 * 8 + kk
                v = idx_v[pl.ds(k * L, L)]
                col = lax.shift_right_logical(v, 7)
                m = (col >= lo_col) & (col < lo_col + CPT)
                cnt = plsc.all_reduce_population_count(m)[0]
                offc = jnp.minimum(off, REQ_CAP - L)
                plsc.store_compressed(req_i.at[pl.ds(offc, L)], v, mask=m)
                plsc.store_compressed(req_t.at[pl.ds(offc, L)],
                                      tag_base + k * L + lane, mask=m)
                off = jnp.minimum(off + cnt, REQ_CAP - L)
            return off
        return lax.fori_loop(0, B // L // 8, chunk8, off0)

    n_req = scan_src(si_hbm, 0, 0)
    n_req = scan_src(oi_hbm, B, n_req)
    n_chunks = (n_req + L - 1) // L

    zeros16 = jnp.zeros((L,), jnp.int32)
    gcounts = zeros16
    for g in range(16):
        def gscan(k4, cur, g=g):
            for kk in range(4):
                k = k4 * 4 + kk
                kb = jnp.minimum(k * L, REQ_CAP - L)
                v = req_i[pl.ds(kb, L)]
                t = req_t[pl.ds(kb, L)]
                col = lax.shift_right_logical(v, 7)
                m = (lax.shift_right_logical(col - lo_col, 4) == g)
                m = m & ((kb + lane) < n_req)
                cnt = plsc.all_reduce_population_count(m)[0]
                curc = jnp.minimum(cur, GCAP - L)
                plsc.store_compressed(req2_i.at[g, pl.ds(curc, L)],
                                      v, mask=m)
                plsc.store_compressed(req2_t.at[g, pl.ds(curc, L)],
                                      t, mask=m)
                cur = jnp.minimum(cur + cnt, GCAP - L)
            return cur
        cur_g = lax.fori_loop(0, (n_chunks + 3) // 4, gscan, 0)
        gcounts = jnp.where(lane == g, cur_g, gcounts)
    gcnt_v[pl.ds(0, L)] = gcounts

    for q in range(12):
        tagbuf[pl.ds(q * L, L)] = sink

    def fetch_super(s_idx, slot):
        scol = lo_col + s_idx * SUP

        @pl.when(scol < NCOLS)
        def _():
            sbase = jnp.minimum(scol, NCOLS - SUP)
            off = pl.multiple_of(sbase * 128, 128)
            pltpu.async_copy(nt_hbm.at[:, pl.ds(off, SUP * 128)],
                             slab.at[slot], sems[slot])

    def drain_super(s_idx, slot):
        scol = lo_col + s_idx * SUP

        @pl.when(scol < NCOLS)
        def _():
            pltpu.make_async_copy(nt_hbm.at[:, pl.ds(0, SUP * 128)],
                                  slab.at[slot], sems[slot]).wait()

    def flush128():
        for q in range(8):
            tagidx[q, pl.ds(0, L)] = tagbuf[pl.ds(q * L, L)]
        cps = []
        for q in range(8):
            cps.append(pltpu.make_async_copy(
                outbuf.at[pl.ds(q * L, L), :],
                stage_hbm.at[tagidx.at[q]], scsem))
        for cp in cps:
            cp.start()
        for cp in cps:
            cp.wait()
        for rr in range(L):
            for j in range(4):
                outbuf[rr, pl.ds(j * L, L)] = outbuf[128 + rr,
                                                     pl.ds(j * L, L)]
        tagbuf[pl.ds(0, L)] = tagbuf[pl.ds(128, L)]
        for q in range(1, 12):
            tagbuf[pl.ds(q * L, L)] = sink

    def process_super(s_idx, slot, ob):
        scol = lo_col + s_idx * SUP
        sbase = jnp.minimum(scol, NCOLS - SUP)
        g = lax.shift_right_logical(s_idx, 2)
        cnt_g = plsc.load_gather(gcnt_v,
                                 [jnp.full((L,), g, jnp.int32)])[0]

        def req_chunk(k, ob):
            v = req2_i[g, pl.ds(k * L, L)]
            t = req2_t[g, pl.ds(k * L, L)]
            col = lax.shift_right_logical(v, 7)
            m = (col >= scol) & (col < scol + SUP)
            m = m & ((k * L + lane) < cnt_g)
            cnt = plsc.all_reduce_population_count(m)[0]

            def do_extract(ob):
                tmp_i[pl.ds(0, L)] = jnp.zeros((L,), jnp.int32)
                tmp_t[pl.ds(0, L)] = sink
                plsc.store_compressed(tmp_i.at[pl.ds(0, L)], v, mask=m)
                plsc.store_compressed(tmp_t.at[pl.ds(0, L)], t, mask=m)
                ti = tmp_i[pl.ds(0, L)]
                tt = tmp_t[pl.ds(0, L)]
                tagbuf[pl.ds(ob, L)] = tt
                for r in range(L):
                    i_r = ti[r]
                    slot_r = jnp.clip(
                        lax.shift_right_logical(i_r, 7) - sbase,
                        0, SUP - 1)
                    il = i_r & 127
                    il_s = jnp.full((L,), slot_r * 128 + il, jnp.int32)
                    for j in range(4):
                        rows = j * L + lane
                        q16 = plsc.load_gather(slab.at[slot],
                                               [rows, il_s])
                        outbuf[ob + r, pl.ds(j * L, L)] = q16
                return ob + cnt

            ob = lax.cond(cnt > 0, do_extract, lambda ob: ob, ob)

            def do_flush(ob):
                flush128()
                return ob - 128
            return lax.cond(ob >= 128, do_flush, lambda ob: ob, ob)

        return lax.fori_loop(0, (cnt_g + L - 1) // L, req_chunk, ob)

    fetch_super(0, 0)

    def pair(u, ob):
        s = 2 * u

        @pl.when(s + 1 < NSUP)
        def _():
            fetch_super(s + 1, 1)
        drain_super(s, 0)
        ob = process_super(s, 0, ob)

        @pl.when(s + 2 < NSUP)
        def _():
            fetch_super(s + 2, 0)
        drain_super(s + 1, 1)
        ob = process_super(s + 1, 1, ob)
        return ob

    out_base = lax.fori_loop(0, NSUP // 2, pair, 0)

    for q in range(8):
        tagidx[q, pl.ds(0, L)] = tagbuf[pl.ds(q * L, L)]
    cps = []
    for q in range(8):
        cps.append(pltpu.make_async_copy(
            outbuf.at[pl.ds(q * L, L), :],
            stage_hbm.at[tagidx.at[q]], scsem))
    for q in range(8):
        @pl.when(q * L < out_base)
        def _():
            cps[q].start()
    for q in range(8):
        @pl.when(q * L < out_base)
        def _():
            cps[q].wait()


def _score_body(stage_hbm, pi_hbm, rt_hbm, out_hbm,
                pi_v, rt_v, s_v, o_v, out_v, shared):
    cid = lax.axis_index("c")
    sid = lax.axis_index("s")
    wid = cid * NUM_SUBCORES + sid
    base = wid * BPW
    lane = lax.iota(jnp.int32, L)
    pltpu.sync_copy(pi_hbm, pi_v)
    pltpu.sync_copy(rt_hbm, rt_v)

    def one_pass(p, _):
        pltpu.sync_copy(stage_hbm.at[pl.ds(base + p * 128, 128), :], s_v)
        pltpu.sync_copy(stage_hbm.at[pl.ds(B + base + p * 128, 128), :],
                        o_v)

        def group(g, _):
            pv = pi_v[pl.ds(base + p * 128 + g * L, L)]
            out16 = jnp.zeros((L,), jnp.float32)
            for r in range(L):
                b = g * L + r
                pv_s = jnp.full((L,), pv[r], jnp.int32)
                acc = jnp.zeros((L,), jnp.float32)
                for j in range(4):
                    dj = j * L + lane
                    pq = plsc.load_gather(rt_v, [dj, pv_s])
                    acc = acc + (s_v[b, pl.ds(j * L, L)]
                                 * o_v[b, pl.ds(j * L, L)] * pq)
                out16 = jnp.where(lane == r, jnp.sum(acc), out16)
            out_v[pl.ds(p * 128 + g * L, L)] = out16
            return 0
        lax.fori_loop(0, 8, group, 0)
        return 0

    lax.fori_loop(0, BPW // 128, one_pass, 0)

    pltpu.sync_copy(out_v, shared.at[pl.ds(sid * BPW, BPW)])
    plsc.subcore_barrier()

    @pl.when(sid == 0)
    def _():
        pltpu.sync_copy(shared,
                        out_hbm.at[pl.ds(cid * NUM_SUBCORES * BPW,
                                         NUM_SUBCORES * BPW)])


@jax.jit
def _distmult(si, pi, oi, nodes, relations):
    mesh = plsc.VectorSubcoreMesh(core_axis_name="c", subcore_axis_name="s")
    nt = jnp.swapaxes(nodes, 0, 1)
    rt = jnp.swapaxes(jnp.pad(relations, ((0, 1024 - R), (0, 0))), 0, 1)
    cp = pltpu.CompilerParams(needs_layout_passes=False)

    stage = pl.kernel(
        _extract_body,
        out_type=jax.ShapeDtypeStruct((STAGE, 128), jnp.float32),
        mesh=mesh,
        scratch_types=[
            pltpu.VMEM((B,), jnp.int32),
            pltpu.VMEM((REQ_CAP,), jnp.int32),
            pltpu.VMEM((REQ_CAP,), jnp.int32),
            pltpu.VMEM((16, GCAP), jnp.int32),
            pltpu.VMEM((16, GCAP), jnp.int32),
            pltpu.VMEM((L,), jnp.int32),
            pltpu.VMEM((L,), jnp.int32),
            pltpu.VMEM((L,), jnp.int32),
            pltpu.VMEM((2, D, SUP * 128), jnp.float32),
            pltpu.VMEM((176, 128), jnp.float32),
            pltpu.VMEM((192,), jnp.int32),
            pltpu.VMEM((8, L), jnp.int32),
            pltpu.SemaphoreType.DMA,
            pltpu.SemaphoreType.DMA,
            pltpu.SemaphoreType.DMA,
            pltpu.SemaphoreType.DMA,
            pltpu.SemaphoreType.DMA,
        ],
        compiler_params=cp,
    )(si, oi, nt)

    return pl.kernel(
        _score_body,
        out_type=jax.ShapeDtypeStruct((B,), jnp.float32),
        mesh=mesh,
        scratch_types=[
            pltpu.VMEM((B,), jnp.int32),
            pltpu.VMEM((D, 1024), jnp.float32),
            pltpu.VMEM((128, 128), jnp.float32),
            pltpu.VMEM((128, 128), jnp.float32),
            pltpu.VMEM((BPW,), jnp.float32),
            pltpu.VMEM_SHARED((NUM_SUBCORES * BPW,), jnp.float32),
        ],
        compiler_params=cp,
    )(stage, pi, rt)


def kernel(si, pi, oi, nodes, relations):
    return _distmult(si.astype(jnp.int32), pi.astype(jnp.int32),
                     oi.astype(jnp.int32), nodes, relations)
